# Initial kernel scaffold; baseline (speedup 1.0000x reference)
import jax
import jax.numpy as jnp
from jax.experimental import pallas as pl


def kernel(x, y, x_weights, y_weights):
    raise NotImplementedError("write your pallas kernel here")


# speedup vs baseline: 2095.6875x; 2095.6875x over previous
import functools

import jax
import jax.numpy as jnp
from jax import lax
from jax.experimental import pallas as pl
from jax.experimental.pallas import tpu as pltpu
from jax.experimental.pallas import tpu_sc as plsc

NSAMP = 1000000
LANES = 16
NTILES = 32
PER_TILE = 64000
PAD_SIDE = 16 * PER_TILE - NSAMP
CHUNK = 4000
NCHUNK = PER_TILE // CHUNK
NVREG = CHUNK // LANES
B = 2048
HIST = LANES * B


def _reduce_body(xk, yk, xw, yw, vmin_o, vmax_o, wx_o, wy_o):
    vmin_o[...] = jnp.minimum(jnp.min(xk[...]), jnp.min(yk[...])).reshape(1, 1)
    vmax_o[...] = jnp.maximum(jnp.max(xk[...]), jnp.max(yk[...])).reshape(1, 1)
    wx_o[...] = jnp.sum(xw[...]).reshape(1, 1)
    wy_o[...] = jnp.sum(yw[...]).reshape(1, 1)


def _reduce_tc(xk, yk, xw, yw):
    s11 = jax.ShapeDtypeStruct((1, 1), jnp.float32)
    return pl.pallas_call(
        _reduce_body,
        out_shape=(s11, s11, s11, s11),
    )(xk, yk, xw, yw)


def _sc_hist_body(keys_hbm, w_hbm, consts_hbm, out_hbm,
                  wsum, iwsum, cvm, kbuf, wbuf):
    cid = lax.axis_index("c")
    sid = lax.axis_index("s")
    wid = sid * 2 + cid
    base = wid * PER_TILE

    pltpu.sync_copy(consts_hbm.at[wid], cvm)
    vmin_v = cvm[0, :]
    h_v = cvm[1, :]
    inv_h_v = cvm[2, :]
    c_v = cvm[3, :]

    lane = lax.iota(jnp.int32, 16)
    lane_off = lane * B
    zeros16 = jnp.zeros((16,), jnp.float32)

    def zero_body(i, _):
        wsum[pl.ds(i * 16, 16)] = zeros16
        iwsum[pl.ds(i * 16, 16)] = zeros16
        return 0

    lax.fori_loop(0, HIST // 16, zero_body, 0)

    for c in range(NCHUNK):
        start = base + c * CHUNK
        pltpu.sync_copy(keys_hbm.at[pl.ds(start, CHUNK)], kbuf)
        pltpu.sync_copy(w_hbm.at[pl.ds(start, CHUNK)], wbuf)

        def body(i, _):
            k = kbuf[pl.ds(i * 16, 16)]
            w = wbuf[pl.ds(i * 16, 16)]
            t = (k - vmin_v) * inv_h_v
            b = jnp.clip(t.astype(jnp.int32), 0, B - 1)
            s = w * c_v
            dd = (b.astype(jnp.float32) + 1.0) * h_v + vmin_v - k
            d = jnp.clip(dd, 0.0, h_v)
            idx = lane_off + b
            plsc.addupdate_scatter(wsum, [idx], s)
            plsc.addupdate_scatter(iwsum, [idx], s * d)
            return 0

        lax.fori_loop(0, NVREG, body, 0)

    pltpu.sync_copy(wsum, out_hbm.at[wid, 0])
    pltpu.sync_copy(iwsum, out_hbm.at[wid, 1])


def _sc_hist(keys, w, consts):
    mesh = plsc.VectorSubcoreMesh(core_axis_name="c", subcore_axis_name="s")
    return pl.kernel(
        _sc_hist_body,
        mesh=mesh,
        compiler_params=pltpu.CompilerParams(needs_layout_passes=False),
        out_type=jax.ShapeDtypeStruct((NTILES, 2, HIST), jnp.float32),
        scratch_types=[
            pltpu.VMEM((HIST,), jnp.float32),
            pltpu.VMEM((HIST,), jnp.float32),
            pltpu.VMEM((4, 16), jnp.float32),
            pltpu.VMEM((CHUNK,), jnp.float32),
            pltpu.VMEM((CHUNK,), jnp.float32),
        ],
    )(keys, w, consts)


def _final_body(hist_ref, h_ref, out_ref):
    a = hist_ref[...]
    ws = jnp.sum(a[:, 0, :, :], axis=(0, 1))
    iw = jnp.sum(a[:, 1, :, :], axis=(0, 1))
    h = h_ref[0, 0]

    ws2 = ws.reshape(16, 128)
    iw2 = iw.reshape(16, 128)

    ii = lax.broadcasted_iota(jnp.int32, (128, 128), 0)
    jj = lax.broadcasted_iota(jnp.int32, (128, 128), 1)
    ltri = jnp.where(ii <= jj, 1.0, 0.0).astype(jnp.float32)
    c1 = jax.lax.dot_general(ws2, ltri, (((1,), (0,)), ((), ())),
                             preferred_element_type=jnp.float32)
    rowtot = c1[:, 127:128]
    ri = lax.broadcasted_iota(jnp.int32, (16, 16), 0)
    rj = lax.broadcasted_iota(jnp.int32, (16, 16), 1)
    stri = jnp.where(ri > rj, 1.0, 0.0).astype(jnp.float32)
    rowoff = jax.lax.dot_general(stri, rowtot, (((1,), (0,)), ((), ())),
                                 preferred_element_type=jnp.float32)
    s_incl = c1 + rowoff
    s0 = s_incl - ws2
    out_ref[...] = jnp.sum(jnp.abs(s0 * h + iw2)).reshape(1, 1)


def _final_tc(hist, h11):
    return pl.pallas_call(
        _final_body,
        out_shape=jax.ShapeDtypeStruct((1, 1), jnp.float32),
    )(hist, h11)


def kernel(x, y, x_weights, y_weights):
    f32 = jnp.float32
    pad64 = jnp.zeros((64,), f32)
    xk2 = jnp.concatenate([x, jnp.full((64,), x[0], f32)]).reshape(7813, 128)
    yk2 = jnp.concatenate([y, jnp.full((64,), y[0], f32)]).reshape(7813, 128)
    xw2 = jnp.concatenate([x_weights, pad64]).reshape(7813, 128)
    yw2 = jnp.concatenate([y_weights, pad64]).reshape(7813, 128)

    vmin, vmax, wx, wy = _reduce_tc(xk2, yk2, xw2, yw2)
    vmin_s = vmin[0, 0]
    vmax_s = vmax[0, 0]
    h = jnp.maximum((vmax_s - vmin_s) / B, 1e-30)
    inv_h = 1.0 / h
    cx = 1.0 / wx[0, 0]
    cy = 1.0 / wy[0, 0]

    padk = jnp.zeros((PAD_SIDE,), f32)
    keys = jnp.concatenate([x, padk, y, padk])
    w = jnp.concatenate([x_weights, padk, -y_weights, padk])

    tile_c = jnp.where(jnp.arange(NTILES) < 16, cx, cy)
    consts = jnp.stack([
        jnp.full((NTILES,), vmin_s),
        jnp.full((NTILES,), h),
        jnp.full((NTILES,), inv_h),
        tile_c,
    ], axis=1)
    consts = jnp.broadcast_to(consts[:, :, None], (NTILES, 4, 16)) + 0.0

    hist = _sc_hist(keys, w, consts)
    hist4 = hist.reshape(NTILES, 2, LANES, B)

    out = _final_tc(hist4, h.reshape(1, 1))
    return out[0, 0]


# speedup vs baseline: 2370.8585x; 1.1313x over previous
import functools

import jax
import jax.numpy as jnp
from jax import lax
from jax.experimental import pallas as pl
from jax.experimental.pallas import tpu as pltpu
from jax.experimental.pallas import tpu_sc as plsc

NSAMP = 1000000
LANES = 16
NTILES = 32
PER_TILE = 64000
PAD_SIDE = 16 * PER_TILE - NSAMP
CHUNK = 8000
NCHUNK = PER_TILE // CHUNK
NVREG = CHUNK // LANES
B = 2048
HIST = LANES * B


def _reduce_body(xk, yk, xw, yw, vmin_o, vmax_o, wx_o, wy_o):
    vmin_o[...] = jnp.minimum(jnp.min(xk[...]), jnp.min(yk[...])).reshape(1, 1)
    vmax_o[...] = jnp.maximum(jnp.max(xk[...]), jnp.max(yk[...])).reshape(1, 1)
    wx_o[...] = jnp.sum(xw[...]).reshape(1, 1)
    wy_o[...] = jnp.sum(yw[...]).reshape(1, 1)


def _reduce_tc(xk, yk, xw, yw):
    s11 = jax.ShapeDtypeStruct((1, 1), jnp.float32)
    return pl.pallas_call(
        _reduce_body,
        out_shape=(s11, s11, s11, s11),
    )(xk, yk, xw, yw)


def _sc_hist_body(keys_hbm, w_hbm, consts_hbm, out_hbm,
                  wsum, iwsum, fw, fi, cvm, kbuf0, kbuf1, wbuf0, wbuf1,
                  ksem, wsem):
    kbufs = (kbuf0, kbuf1)
    wbufs = (wbuf0, wbuf1)
    cid = lax.axis_index("c")
    sid = lax.axis_index("s")
    wid = sid * 2 + cid
    base = wid * PER_TILE

    pltpu.sync_copy(consts_hbm.at[wid], cvm)
    vmin_v = cvm[0, :]
    h_v = cvm[1, :]
    inv_h_v = cvm[2, :]
    c_v = cvm[3, :]

    lane = lax.iota(jnp.int32, 16)
    lane_off = lane * B
    zeros16 = jnp.zeros((16,), jnp.float32)

    def zero_body(i, _):
        wsum[pl.ds(i * 16, 16)] = zeros16
        iwsum[pl.ds(i * 16, 16)] = zeros16
        return 0

    lax.fori_loop(0, HIST // 16, zero_body, 0)

    def start_dma(c, slot):
        st = base + c * CHUNK
        ck = pltpu.async_copy(keys_hbm.at[pl.ds(st, CHUNK)],
                              kbufs[slot], ksem.at[slot])
        cw = pltpu.async_copy(w_hbm.at[pl.ds(st, CHUNK)],
                              wbufs[slot], wsem.at[slot])
        return ck, cw

    pend = {0: start_dma(0, 0)}
    for c in range(NCHUNK):
        slot = c % 2
        if c + 1 < NCHUNK:
            pend[(c + 1) % 2] = start_dma(c + 1, (c + 1) % 2)
        ck, cw = pend[slot]
        ck.wait()
        cw.wait()

        kb = kbufs[slot]
        wb = wbufs[slot]

        def body(i, _):
            k = kb[pl.ds(i * 16, 16)]
            w = wb[pl.ds(i * 16, 16)]
            t = (k - vmin_v) * inv_h_v
            b = jnp.clip(t.astype(jnp.int32), 0, B - 1)
            s = w * c_v
            dd = (b.astype(jnp.float32) + 1.0) * h_v + vmin_v - k
            d = jnp.clip(dd, 0.0, h_v)
            idx = lane_off + b
            plsc.addupdate_scatter(wsum, [idx], s)
            plsc.addupdate_scatter(iwsum, [idx], s * d)
            return 0

        lax.fori_loop(0, NVREG, body, 0)

    def fold_body(g, _):
        accw = wsum[pl.ds(g * 16, 16)]
        acci = iwsum[pl.ds(g * 16, 16)]
        for l in range(1, LANES):
            accw = accw + wsum[pl.ds(l * B + g * 16, 16)]
            acci = acci + iwsum[pl.ds(l * B + g * 16, 16)]
        fw[pl.ds(g * 16, 16)] = accw
        fi[pl.ds(g * 16, 16)] = acci
        return 0

    lax.fori_loop(0, B // 16, fold_body, 0)

    pltpu.sync_copy(fw, out_hbm.at[wid, 0])
    pltpu.sync_copy(fi, out_hbm.at[wid, 1])


def _sc_hist(keys, w, consts):
    mesh = plsc.VectorSubcoreMesh(core_axis_name="c", subcore_axis_name="s")
    return pl.kernel(
        _sc_hist_body,
        mesh=mesh,
        compiler_params=pltpu.CompilerParams(needs_layout_passes=False),
        out_type=jax.ShapeDtypeStruct((NTILES, 2, B), jnp.float32),
        scratch_types=[
            pltpu.VMEM((HIST,), jnp.float32),
            pltpu.VMEM((HIST,), jnp.float32),
            pltpu.VMEM((B,), jnp.float32),
            pltpu.VMEM((B,), jnp.float32),
            pltpu.VMEM((4, 16), jnp.float32),
            pltpu.VMEM((CHUNK,), jnp.float32),
            pltpu.VMEM((CHUNK,), jnp.float32),
            pltpu.VMEM((CHUNK,), jnp.float32),
            pltpu.VMEM((CHUNK,), jnp.float32),
            pltpu.SemaphoreType.DMA((2,)),
            pltpu.SemaphoreType.DMA((2,)),
        ],
    )(keys, w, consts)


def _final_body(hist_ref, h_ref, out_ref):
    a = hist_ref[...]
    ws = jnp.sum(a[:, 0, :], axis=0)
    iw = jnp.sum(a[:, 1, :], axis=0)
    h = h_ref[0, 0]

    ws2 = ws.reshape(16, 128)
    iw2 = iw.reshape(16, 128)

    ii = lax.broadcasted_iota(jnp.int32, (128, 128), 0)
    jj = lax.broadcasted_iota(jnp.int32, (128, 128), 1)
    ltri = jnp.where(ii <= jj, 1.0, 0.0).astype(jnp.float32)
    c1 = jax.lax.dot_general(ws2, ltri, (((1,), (0,)), ((), ())),
                             preferred_element_type=jnp.float32)
    rowtot = c1[:, 127:128]
    ri = lax.broadcasted_iota(jnp.int32, (16, 16), 0)
    rj = lax.broadcasted_iota(jnp.int32, (16, 16), 1)
    stri = jnp.where(ri > rj, 1.0, 0.0).astype(jnp.float32)
    rowoff = jax.lax.dot_general(stri, rowtot, (((1,), (0,)), ((), ())),
                                 preferred_element_type=jnp.float32)
    s_incl = c1 + rowoff
    s0 = s_incl - ws2
    out_ref[...] = jnp.sum(jnp.abs(s0 * h + iw2)).reshape(1, 1)


def _final_tc(hist, h11):
    return pl.pallas_call(
        _final_body,
        out_shape=jax.ShapeDtypeStruct((1, 1), jnp.float32),
    )(hist, h11)


def kernel(x, y, x_weights, y_weights):
    f32 = jnp.float32
    pad64 = jnp.zeros((64,), f32)
    xk2 = jnp.concatenate([x, jnp.full((64,), x[0], f32)]).reshape(7813, 128)
    yk2 = jnp.concatenate([y, jnp.full((64,), y[0], f32)]).reshape(7813, 128)
    xw2 = jnp.concatenate([x_weights, pad64]).reshape(7813, 128)
    yw2 = jnp.concatenate([y_weights, pad64]).reshape(7813, 128)

    vmin, vmax, wx, wy = _reduce_tc(xk2, yk2, xw2, yw2)
    vmin_s = vmin[0, 0]
    vmax_s = vmax[0, 0]
    h = jnp.maximum((vmax_s - vmin_s) / B, 1e-30)
    inv_h = 1.0 / h
    cx = 1.0 / wx[0, 0]
    cy = 1.0 / wy[0, 0]

    padk = jnp.zeros((PAD_SIDE,), f32)
    keys = jnp.concatenate([x, padk, y, padk])
    w = jnp.concatenate([x_weights, padk, -y_weights, padk])

    tile_c = jnp.where(jnp.arange(NTILES) < 16, cx, cy)
    consts = jnp.stack([
        jnp.full((NTILES,), vmin_s),
        jnp.full((NTILES,), h),
        jnp.full((NTILES,), inv_h),
        tile_c,
    ], axis=1)
    consts = jnp.broadcast_to(consts[:, :, None], (NTILES, 4, 16)) + 0.0

    hist = _sc_hist(keys, w, consts)

    out = _final_tc(hist, h.reshape(1, 1))
    return out[0, 0]


# speedup vs baseline: 2411.8967x; 1.0173x over previous
import functools

import jax
import jax.numpy as jnp
from jax import lax
from jax.experimental import pallas as pl
from jax.experimental.pallas import tpu as pltpu
from jax.experimental.pallas import tpu_sc as plsc

NSAMP = 1000000
LANES = 16
NTILES = 32
PER_TILE = 64000
PAD_SIDE = 16 * PER_TILE - NSAMP
CHUNK = 8000
NCHUNK = PER_TILE // CHUNK
NVREG = CHUNK // LANES
B = 2048
HIST = LANES * B


def _reduce_body(xk, yk, xw, yw, vmin_o, vmax_o, wx_o, wy_o):
    vmin_o[...] = jnp.minimum(jnp.min(xk[...]), jnp.min(yk[...])).reshape(1, 1)
    vmax_o[...] = jnp.maximum(jnp.max(xk[...]), jnp.max(yk[...])).reshape(1, 1)
    wx_o[...] = jnp.sum(xw[...]).reshape(1, 1)
    wy_o[...] = jnp.sum(yw[...]).reshape(1, 1)


def _reduce_tc(xk, yk, xw, yw):
    s11 = jax.ShapeDtypeStruct((1, 1), jnp.float32)
    return pl.pallas_call(
        _reduce_body,
        out_shape=(s11, s11, s11, s11),
    )(xk, yk, xw, yw)


def _sc_hist_body(keys_hbm, w_hbm, consts_hbm, out_hbm,
                  wsum, iwsum, fw, fi, cvm, kbuf0, kbuf1, wbuf0, wbuf1,
                  ksem, wsem):
    kbufs = (kbuf0, kbuf1)
    wbufs = (wbuf0, wbuf1)
    cid = lax.axis_index("c")
    sid = lax.axis_index("s")
    wid = sid * 2 + cid
    base = wid * PER_TILE

    pltpu.sync_copy(consts_hbm.at[wid], cvm)
    vmin_v = cvm[0, :]
    h_v = cvm[1, :]
    inv_h_v = cvm[2, :]
    c_v = cvm[3, :]

    lane = lax.iota(jnp.int32, 16)
    lane_off = lane * B
    zeros16 = jnp.zeros((16,), jnp.float32)

    def zero_body(i, _):
        for u in range(8):
            wsum[pl.ds(i * 128 + u * 16, 16)] = zeros16
            iwsum[pl.ds(i * 128 + u * 16, 16)] = zeros16
        return 0

    lax.fori_loop(0, HIST // 128, zero_body, 0)

    def start_dma(c, slot):
        st = base + c * CHUNK
        ck = pltpu.async_copy(keys_hbm.at[pl.ds(st, CHUNK)],
                              kbufs[slot], ksem.at[slot])
        cw = pltpu.async_copy(w_hbm.at[pl.ds(st, CHUNK)],
                              wbufs[slot], wsem.at[slot])
        return ck, cw

    pend = {0: start_dma(0, 0)}
    for c in range(NCHUNK):
        slot = c % 2
        if c + 1 < NCHUNK:
            pend[(c + 1) % 2] = start_dma(c + 1, (c + 1) % 2)
        ck, cw = pend[slot]
        ck.wait()
        cw.wait()

        kb = kbufs[slot]
        wb = wbufs[slot]

        def body(i, _):
            for u in range(4):
                k = kb[pl.ds(i * 64 + u * 16, 16)]
                w = wb[pl.ds(i * 64 + u * 16, 16)]
                t = (k - vmin_v) * inv_h_v
                b = jnp.clip(t.astype(jnp.int32), 0, B - 1)
                s = w * c_v
                dd = (b.astype(jnp.float32) + 1.0) * h_v + vmin_v - k
                d = jnp.clip(dd, 0.0, h_v)
                idx = lane_off + b
                plsc.addupdate_scatter(wsum, [idx], s)
                plsc.addupdate_scatter(iwsum, [idx], s * d)
            return 0

        lax.fori_loop(0, NVREG // 4, body, 0)

    def fold_body(g, _):
        accw = wsum[pl.ds(g * 16, 16)]
        acci = iwsum[pl.ds(g * 16, 16)]
        for l in range(1, LANES):
            accw = accw + wsum[pl.ds(l * B + g * 16, 16)]
            acci = acci + iwsum[pl.ds(l * B + g * 16, 16)]
        fw[pl.ds(g * 16, 16)] = accw
        fi[pl.ds(g * 16, 16)] = acci
        return 0

    lax.fori_loop(0, B // 16, fold_body, 0)

    pltpu.sync_copy(fw, out_hbm.at[wid, 0])
    pltpu.sync_copy(fi, out_hbm.at[wid, 1])


def _sc_hist(keys, w, consts):
    mesh = plsc.VectorSubcoreMesh(core_axis_name="c", subcore_axis_name="s")
    return pl.kernel(
        _sc_hist_body,
        mesh=mesh,
        compiler_params=pltpu.CompilerParams(needs_layout_passes=False),
        out_type=jax.ShapeDtypeStruct((NTILES, 2, B), jnp.float32),
        scratch_types=[
            pltpu.VMEM((HIST,), jnp.float32),
            pltpu.VMEM((HIST,), jnp.float32),
            pltpu.VMEM((B,), jnp.float32),
            pltpu.VMEM((B,), jnp.float32),
            pltpu.VMEM((4, 16), jnp.float32),
            pltpu.VMEM((CHUNK,), jnp.float32),
            pltpu.VMEM((CHUNK,), jnp.float32),
            pltpu.VMEM((CHUNK,), jnp.float32),
            pltpu.VMEM((CHUNK,), jnp.float32),
            pltpu.SemaphoreType.DMA((2,)),
            pltpu.SemaphoreType.DMA((2,)),
        ],
    )(keys, w, consts)


def _final_body(hist_ref, h_ref, out_ref):
    a = hist_ref[...]
    ws = jnp.sum(a[:, 0, :], axis=0)
    iw = jnp.sum(a[:, 1, :], axis=0)
    h = h_ref[0, 0]

    ws2 = ws.reshape(16, 128)
    iw2 = iw.reshape(16, 128)

    ii = lax.broadcasted_iota(jnp.int32, (128, 128), 0)
    jj = lax.broadcasted_iota(jnp.int32, (128, 128), 1)
    ltri = jnp.where(ii <= jj, 1.0, 0.0).astype(jnp.float32)
    c1 = jax.lax.dot_general(ws2, ltri, (((1,), (0,)), ((), ())),
                             preferred_element_type=jnp.float32)
    rowtot = c1[:, 127:128]
    ri = lax.broadcasted_iota(jnp.int32, (16, 16), 0)
    rj = lax.broadcasted_iota(jnp.int32, (16, 16), 1)
    stri = jnp.where(ri > rj, 1.0, 0.0).astype(jnp.float32)
    rowoff = jax.lax.dot_general(stri, rowtot, (((1,), (0,)), ((), ())),
                                 preferred_element_type=jnp.float32)
    s_incl = c1 + rowoff
    s0 = s_incl - ws2
    out_ref[...] = jnp.sum(jnp.abs(s0 * h + iw2)).reshape(1, 1)


def _final_tc(hist, h11):
    return pl.pallas_call(
        _final_body,
        out_shape=jax.ShapeDtypeStruct((1, 1), jnp.float32),
    )(hist, h11)


def kernel(x, y, x_weights, y_weights):
    f32 = jnp.float32
    pad64 = jnp.zeros((64,), f32)
    xk2 = jnp.concatenate([x, jnp.full((64,), x[0], f32)]).reshape(7813, 128)
    yk2 = jnp.concatenate([y, jnp.full((64,), y[0], f32)]).reshape(7813, 128)
    xw2 = jnp.concatenate([x_weights, pad64]).reshape(7813, 128)
    yw2 = jnp.concatenate([y_weights, pad64]).reshape(7813, 128)

    vmin, vmax, wx, wy = _reduce_tc(xk2, yk2, xw2, yw2)
    vmin_s = vmin[0, 0]
    vmax_s = vmax[0, 0]
    h = jnp.maximum((vmax_s - vmin_s) / B, 1e-30)
    inv_h = 1.0 / h
    cx = 1.0 / wx[0, 0]
    cy = 1.0 / wy[0, 0]

    padk = jnp.zeros((PAD_SIDE,), f32)
    keys = jnp.concatenate([x, padk, y, padk])
    w = jnp.concatenate([x_weights, padk, -y_weights, padk])

    tile_c = jnp.where(jnp.arange(NTILES) < 16, cx, cy)
    consts = jnp.stack([
        jnp.full((NTILES,), vmin_s),
        jnp.full((NTILES,), h),
        jnp.full((NTILES,), inv_h),
        tile_c,
    ], axis=1)
    consts = jnp.broadcast_to(consts[:, :, None], (NTILES, 4, 16)) + 0.0

    hist = _sc_hist(keys, w, consts)

    out = _final_tc(hist, h.reshape(1, 1))
    return out[0, 0]
